# baseline (speedup 1.0000x reference)
import functools

import jax
import jax.numpy as jnp
from jax import lax
from jax.experimental import pallas as pl
from jax.experimental.pallas import tpu as pltpu
from jax.experimental.pallas import tpu_sc as plsc

N = 10000
D = 128
E = 320000

NW = 32
NPW = 320
NPAD = NW * NPW
C = 2560
NCHUNK = E // C
C_G = C // 16
F = 128
LCAP = E + F
LBUF = C + F + 16

_mesh = plsc.VectorSubcoreMesh(core_axis_name="c", subcore_axis_name="s")
_params = pltpu.CompilerParams(needs_layout_passes=False)


def _wid():
    return lax.axis_index("s") * 2 + lax.axis_index("c")


@functools.partial(
    pl.kernel,
    out_type=(
        jax.ShapeDtypeStruct((NW * LCAP,), jnp.int32),
        jax.ShapeDtypeStruct((NW * LCAP,), jnp.int32),
        jax.ShapeDtypeStruct((NW * 16,), jnp.int32),
    ),
    mesh=_mesh,
    compiler_params=_params,
    scratch_types=[
        pltpu.VMEM((C,), jnp.int32),
        pltpu.VMEM((C,), jnp.int32),
        pltpu.VMEM((C,), jnp.int32),
        pltpu.VMEM((C,), jnp.int32),
        pltpu.VMEM((LBUF,), jnp.int32),
        pltpu.VMEM((LBUF,), jnp.int32),
        pltpu.VMEM((16,), jnp.int32),
        pltpu.SemaphoreType.DMA,
        pltpu.SemaphoreType.DMA,
        pltpu.SemaphoreType.DMA,
        pltpu.SemaphoreType.DMA,
    ],
)
def _phase0(src_hbm, dst_hbm, slist, dlist, nblk,
            sb0, db0, sb1, db1, lsrc, ldst, cntv,
            ss0, sd0, ss1, sd1):
    w = _wid()
    lo = w * NPW
    iota16 = lax.iota(jnp.int32, 16)
    zero16 = jnp.zeros((16,), jnp.int32)
    one16 = jnp.ones((16,), jnp.int32)

    def start_chunk(c, sb, db, ss, sd):
        pltpu.async_copy(src_hbm.at[pl.ds(c * C, C)], sb, ss)
        pltpu.async_copy(dst_hbm.at[pl.ds(c * C, C)], db, sd)

    def wait_chunk(sb, db, ss, sd):
        pltpu.make_async_copy(src_hbm.at[pl.ds(0, C)], sb, ss).wait()
        pltpu.make_async_copy(dst_hbm.at[pl.ds(0, C)], db, sd).wait()

    def do_chunk(c, carry, sb, db, ss, sd, sb_o, db_o, ss_o, sd_o):
        ptr, total = carry

        def group_body(q, p):
            lov = zero16 + lo
            dv = [db[pl.ds((8 * q + i) * 16, 16)] for i in range(8)]
            sv = [sb[pl.ds((8 * q + i) * 16, 16)] for i in range(8)]
            ms = [(d >= lov) & (d < lov + NPW) for d in dv]
            cnts = [plsc.all_reduce_population_count(m)[0] for m in ms]
            mis = [jnp.where(m, one16, zero16) for m in ms]
            cums = [plsc.cumsum(mi) for mi in mis]
            bases = [p]
            for i in range(7):
                bases.append(bases[i] + cnts[i])
            for i in range(8):
                idx = (zero16 + bases[i]) + cums[i] - mis[i]
                plsc.store_scatter(lsrc, [idx], sv[i], mask=ms[i])
                plsc.store_scatter(ldst, [idx], dv[i], mask=ms[i])
            return bases[7] + cnts[7]

        ptr = lax.fori_loop(0, C_G // 8, group_body, ptr)
        nfull = ptr // F

        def flush_body(i, _):
            off = pl.multiple_of(w * LCAP + total + i * F, 8)
            pltpu.sync_copy(lsrc.at[pl.ds(i * F, F)], slist.at[pl.ds(off, F)])
            pltpu.sync_copy(ldst.at[pl.ds(i * F, F)], dlist.at[pl.ds(off, F)])
            return 0

        lax.fori_loop(0, nfull, flush_body, 0)
        base = nfull * F
        for j in range(F // 16 + 1):
            vs = lsrc[pl.ds(base + j * 16, 16)]
            vd = ldst[pl.ds(base + j * 16, 16)]
            lsrc[pl.ds(j * 16, 16)] = vs
            ldst[pl.ds(j * 16, 16)] = vd

        @pl.when(c + 1 < NCHUNK)
        def _():
            wait_chunk(sb_o, db_o, ss_o, sd_o)

        @pl.when(c + 2 < NCHUNK)
        def _():
            start_chunk(c + 2, sb, db, ss, sd)

        return ptr - base, total + base

    def chunk_body(c, carry):
        def even():
            return do_chunk(c, carry, sb0, db0, ss0, sd0, sb1, db1, ss1, sd1)

        def odd():
            return do_chunk(c, carry, sb1, db1, ss1, sd1, sb0, db0, ss0, sd0)

        return lax.cond(c % 2 == 0, even, odd)

    pltpu.sync_copy(src_hbm.at[pl.ds(0, C)], sb0)
    pltpu.sync_copy(dst_hbm.at[pl.ds(0, C)], db0)
    start_chunk(1, sb1, db1, ss1, sd1)
    ptr, total = lax.fori_loop(0, NCHUNK, chunk_body,
                               (jnp.int32(0), jnp.int32(0)))

    for j in range(F // 16):
        pos = iota16 + j * 16
        vs = lsrc[pl.ds(j * 16, 16)]
        vd = ldst[pl.ds(j * 16, 16)]
        lsrc[pl.ds(j * 16, 16)] = jnp.where(pos >= ptr, 0, vs)
        ldst[pl.ds(j * 16, 16)] = jnp.where(pos >= ptr, lo + NPW, vd)
    foff = pl.multiple_of(w * LCAP + total, 8)
    pltpu.sync_copy(lsrc.at[pl.ds(0, F)], slist.at[pl.ds(foff, F)])
    pltpu.sync_copy(ldst.at[pl.ds(0, F)], dlist.at[pl.ds(foff, F)])

    cntv[...] = jnp.zeros((16,), jnp.int32) + (total // F + 1)
    pltpu.sync_copy(cntv, nblk.at[pl.ds(pl.multiple_of(w * 16, 8), 16)])


@functools.partial(
    pl.kernel,
    out_type=jax.ShapeDtypeStruct((N, D), jnp.float32),
    mesh=_mesh,
    compiler_params=_params,
    scratch_types=[
        pltpu.VMEM((F,), jnp.int32),
        pltpu.VMEM((F,), jnp.int32),
        pltpu.VMEM((F,), jnp.int32),
        pltpu.VMEM((F,), jnp.int32),
        pltpu.VMEM((F,), jnp.int32),
        pltpu.VMEM((F,), jnp.int32),
        pltpu.VMEM((F,), jnp.int32),
        pltpu.VMEM((F, D), jnp.float32),
        pltpu.VMEM((F, D), jnp.float32),
        pltpu.VMEM((F, D), jnp.float32),
        pltpu.VMEM((NPW + 1, D), jnp.float32),
        pltpu.VMEM((16,), jnp.int32),
        pltpu.SemaphoreType.DMA,
        pltpu.SemaphoreType.DMA,
        pltpu.SemaphoreType.DMA,
        pltpu.SemaphoreType.DMA,
        pltpu.SemaphoreType.DMA,
        pltpu.SemaphoreType.DMA,
        pltpu.SemaphoreType.DMA,
        pltpu.SemaphoreType.DMA,
        pltpu.SemaphoreType.DMA,
    ],
)
def _segmax(xin, slist, dlist, nblk, agg,
            sb0, sb1, sb2, db0, db1, db2, dcomp, rw0, rw1, rw2, acc, cntv,
            ss0, ss1, ss2, sd0, sd1, sd2, sg0, sg1, sg2):
    w = _wid()
    lo = w * NPW
    ninf = jnp.full((16,), -jnp.inf, jnp.float32)

    def init_body(r, _):
        for k in range(D // 16):
            acc[r, pl.ds(k * 16, 16)] = ninf
        return 0

    lax.fori_loop(0, NPW + 1, init_body, 0)

    pltpu.sync_copy(nblk.at[pl.ds(pl.multiple_of(w * 16, 8), 16)], cntv)
    nb = jnp.max(cntv[...])

    def start_idx(b, sb, db, ss, sd):
        off = pl.multiple_of(w * LCAP + b * F, 8)
        pltpu.async_copy(slist.at[pl.ds(off, F)], sb, ss)
        pltpu.async_copy(dlist.at[pl.ds(off, F)], db, sd)

    def wait_idx(sb, db, ss, sd):
        pltpu.make_async_copy(slist.at[pl.ds(0, F)], sb, ss).wait()
        pltpu.make_async_copy(dlist.at[pl.ds(0, F)], db, sd).wait()

    def stage(b, cur, nxt, nx2):
        sb, db, rw, ss, sd, sg = cur
        sb2_, db2_, rw2_, ss2_, sd2_, sg2_ = nx2

        @pl.when(b + 2 < nb)
        def _():
            wait_idx(sb2_, db2_, ss2_, sd2_)
            pltpu.async_copy(xin.at[sb2_], rw2_, sg2_)

        pltpu.make_async_copy(xin.at[sb], rw, sg).wait()
        for j in range(F // 16):
            dcomp[pl.ds(j * 16, 16)] = db[pl.ds(j * 16, 16)]

        @pl.when(b + 3 < nb)
        def _():
            start_idx(b + 3, sb, db, ss, sd)

        def group_body(g, _):
            dloc = dcomp[pl.ds(g * 32, 16)] - lo
            dloc2 = dcomp[pl.ds(g * 32 + 16, 16)] - lo
            for l in range(32):
                e = g * 32 + l
                dl = dloc[l] if l < 16 else dloc2[l - 16]
                avs = [acc[dl, pl.ds(k * 16, 16)] for k in range(D // 16)]
                rvs = [rw[e, pl.ds(k * 16, 16)] for k in range(D // 16)]
                for k in range(D // 16):
                    acc[dl, pl.ds(k * 16, 16)] = jnp.maximum(avs[k], rvs[k])
            return 0

        lax.fori_loop(0, F // 32, group_body, 0)

    bufs = [
        (sb0, db0, rw0, ss0, sd0, sg0),
        (sb1, db1, rw1, ss1, sd1, sg1),
        (sb2, db2, rw2, ss2, sd2, sg2),
    ]

    def block_body(b, _):
        for r in range(3):
            @pl.when(b % 3 == r)
            def _(r=r):
                stage(b, bufs[r], bufs[(r + 1) % 3], bufs[(r + 2) % 3])

        return 0

    off0 = pl.multiple_of(w * LCAP, 8)
    pltpu.sync_copy(slist.at[pl.ds(off0, F)], sb0)
    pltpu.sync_copy(dlist.at[pl.ds(off0, F)], db0)
    pltpu.async_copy(xin.at[sb0], rw0, sg0)

    @pl.when(nb > 1)
    def _():
        off1 = pl.multiple_of(w * LCAP + F, 8)
        pltpu.sync_copy(slist.at[pl.ds(off1, F)], sb1)
        pltpu.sync_copy(dlist.at[pl.ds(off1, F)], db1)
        pltpu.async_copy(xin.at[sb1], rw1, sg1)

    @pl.when(nb > 2)
    def _():
        start_idx(2, sb2, db2, ss2, sd2)

    lax.fori_loop(0, nb, block_body, 0)

    def out_body(r, _):
        vs = [acc[r, pl.ds(k * 16, 16)] for k in range(D // 16)]
        for k in range(D // 16):
            acc[r, pl.ds(k * 16, 16)] = jnp.where(vs[k] == -jnp.inf, 0.0,
                                                  vs[k])
        return 0

    lax.fori_loop(0, NPW, out_body, 0)

    @pl.when(w < NW - 1)
    def _():
        pltpu.sync_copy(acc.at[pl.ds(0, NPW), :], agg.at[pl.ds(lo, NPW), :])

    @pl.when(w == NW - 1)
    def _():
        pltpu.sync_copy(acc.at[pl.ds(0, N - (NW - 1) * NPW), :],
                        agg.at[pl.ds(lo, N - (NW - 1) * NPW), :])


_BR = 1000


def _mm(a, b):
    return lax.dot_general(a, b, (((1,), (0,)), ((), ())),
                           preferred_element_type=jnp.float32)


def _elu(z):
    return jnp.where(z > 0, z, jnp.exp(jnp.minimum(z, 0.0)) - 1.0)


def _dense_body(agg_ref, h_ref, wl_ref, wr_ref, b_ref, out_ref, *, res):
    z = _mm(agg_ref[...], wl_ref[...]) + _mm(h_ref[...], wr_ref[...]) + b_ref[...]
    h = _elu(z)
    if res:
        h = h + h_ref[...]
    out_ref[...] = h


def _final_body(agg_ref, h_ref, wl_ref, wr_ref, b_ref, wlin_ref, blin_ref,
                out_ref):
    z = _mm(agg_ref[...], wl_ref[...]) + _mm(h_ref[...], wr_ref[...]) + b_ref[...]
    h3 = _elu(z) + h_ref[...]
    t = _mm(h3, wlin_ref[...]) + blin_ref[...]
    out_ref[...] = 1.0 / (1.0 + jnp.exp(-t))


_row_spec = pl.BlockSpec((_BR, D), lambda i: (i, 0))
_w_spec = pl.BlockSpec((D, D), lambda i: (0, 0))
_b_spec = pl.BlockSpec((1, D), lambda i: (0, 0))


def _dense(agg, h, wl, wr, b, res):
    return pl.pallas_call(
        functools.partial(_dense_body, res=res),
        grid=(N // _BR,),
        in_specs=[_row_spec, _row_spec, _w_spec, _w_spec, _b_spec],
        out_specs=_row_spec,
        out_shape=jax.ShapeDtypeStruct((N, D), jnp.float32),
    )(agg, h, wl, wr, b.reshape(1, D))


def _final(agg, h, wl, wr, b, wlin_pad, blin):
    return pl.pallas_call(
        _final_body,
        grid=(N // _BR,),
        in_specs=[_row_spec, _row_spec, _w_spec, _w_spec, _b_spec, _w_spec,
                  _b_spec],
        out_specs=_row_spec,
        out_shape=jax.ShapeDtypeStruct((N, D), jnp.float32),
    )(agg, h, wl, wr, b.reshape(1, D), wlin_pad, blin)


def kernel(x, edge_index, W1l, W1r, b1, W2l, W2r, b2, W3l, W3r, b3, Wlin,
           blin):
    src = edge_index[0]
    dst = edge_index[1]
    wlin_pad = jnp.zeros((D, D), jnp.float32).at[:, :1].set(Wlin)
    blin_pad = jnp.zeros((1, D), jnp.float32) + blin

    slist, dlist, nblk = _phase0(src, dst)

    agg1 = _segmax(x, slist, dlist, nblk)
    h1 = _dense(agg1, x, W1l, W1r, b1, res=False)
    agg2 = _segmax(h1, slist, dlist, nblk)
    h2 = _dense(agg2, h1, W2l, W2r, b2, res=True)
    agg3 = _segmax(h2, slist, dlist, nblk)
    out = _final(agg3, h2, W3l, W3r, b3, wlin_pad, blin_pad)
    return out[:N, :1]


# speedup vs baseline: 1.0632x; 1.0632x over previous
import functools

import jax
import jax.numpy as jnp
from jax import lax
from jax.experimental import pallas as pl
from jax.experimental.pallas import tpu as pltpu
from jax.experimental.pallas import tpu_sc as plsc

N = 10000
D = 128
E = 320000

NW = 32
NPW = 320
NPAD = NW * NPW
C = 2560
NCHUNK = E // C
C_G = C // 16
F = 128
LCAP = E + F
LBUF = C + F + 16

_mesh = plsc.VectorSubcoreMesh(core_axis_name="c", subcore_axis_name="s")
_params = pltpu.CompilerParams(needs_layout_passes=False)


def _wid():
    return lax.axis_index("s") * 2 + lax.axis_index("c")


@functools.partial(
    pl.kernel,
    out_type=(
        jax.ShapeDtypeStruct((NW * LCAP,), jnp.int32),
        jax.ShapeDtypeStruct((NW * LCAP,), jnp.int32),
        jax.ShapeDtypeStruct((NW * 16,), jnp.int32),
    ),
    mesh=_mesh,
    compiler_params=_params,
    scratch_types=[
        pltpu.VMEM((C,), jnp.int32),
        pltpu.VMEM((C,), jnp.int32),
        pltpu.VMEM((C,), jnp.int32),
        pltpu.VMEM((C,), jnp.int32),
        pltpu.VMEM((LBUF,), jnp.int32),
        pltpu.VMEM((LBUF,), jnp.int32),
        pltpu.VMEM((16,), jnp.int32),
        pltpu.SemaphoreType.DMA,
        pltpu.SemaphoreType.DMA,
        pltpu.SemaphoreType.DMA,
        pltpu.SemaphoreType.DMA,
    ],
)
def _phase0(src_hbm, dst_hbm, slist, dlist, nblk,
            sb0, db0, sb1, db1, lsrc, ldst, cntv,
            ss0, sd0, ss1, sd1):
    w = _wid()
    lo = w * NPW
    iota16 = lax.iota(jnp.int32, 16)
    zero16 = jnp.zeros((16,), jnp.int32)
    one16 = jnp.ones((16,), jnp.int32)

    def start_chunk(c, sb, db, ss, sd):
        pltpu.async_copy(src_hbm.at[pl.ds(c * C, C)], sb, ss)
        pltpu.async_copy(dst_hbm.at[pl.ds(c * C, C)], db, sd)

    def wait_chunk(sb, db, ss, sd):
        pltpu.make_async_copy(src_hbm.at[pl.ds(0, C)], sb, ss).wait()
        pltpu.make_async_copy(dst_hbm.at[pl.ds(0, C)], db, sd).wait()

    def do_chunk(c, carry, sb, db, ss, sd, sb_o, db_o, ss_o, sd_o):
        ptr, total = carry

        def group_body(q, p):
            lov = zero16 + lo
            dv = [db[pl.ds((8 * q + i) * 16, 16)] for i in range(8)]
            sv = [sb[pl.ds((8 * q + i) * 16, 16)] for i in range(8)]
            ms = [(d >= lov) & (d < lov + NPW) for d in dv]
            cnts = [plsc.all_reduce_population_count(m)[0] for m in ms]
            mis = [jnp.where(m, one16, zero16) for m in ms]
            cums = [plsc.cumsum(mi) for mi in mis]
            bases = [p]
            for i in range(7):
                bases.append(bases[i] + cnts[i])
            for i in range(8):
                idx = (zero16 + bases[i]) + cums[i] - mis[i]
                plsc.store_scatter(lsrc, [idx], sv[i], mask=ms[i])
                plsc.store_scatter(ldst, [idx], dv[i], mask=ms[i])
            return bases[7] + cnts[7]

        ptr = lax.fori_loop(0, C_G // 8, group_body, ptr)
        nfull = ptr // F

        def flush_body(i, _):
            off = pl.multiple_of(w * LCAP + total + i * F, 8)
            pltpu.sync_copy(lsrc.at[pl.ds(i * F, F)], slist.at[pl.ds(off, F)])
            pltpu.sync_copy(ldst.at[pl.ds(i * F, F)], dlist.at[pl.ds(off, F)])
            return 0

        lax.fori_loop(0, nfull, flush_body, 0)
        base = nfull * F
        for j in range(F // 16 + 1):
            vs = lsrc[pl.ds(base + j * 16, 16)]
            vd = ldst[pl.ds(base + j * 16, 16)]
            lsrc[pl.ds(j * 16, 16)] = vs
            ldst[pl.ds(j * 16, 16)] = vd

        @pl.when(c + 1 < NCHUNK)
        def _():
            wait_chunk(sb_o, db_o, ss_o, sd_o)

        @pl.when(c + 2 < NCHUNK)
        def _():
            start_chunk(c + 2, sb, db, ss, sd)

        return ptr - base, total + base

    def chunk_body(c, carry):
        def even():
            return do_chunk(c, carry, sb0, db0, ss0, sd0, sb1, db1, ss1, sd1)

        def odd():
            return do_chunk(c, carry, sb1, db1, ss1, sd1, sb0, db0, ss0, sd0)

        return lax.cond(c % 2 == 0, even, odd)

    pltpu.sync_copy(src_hbm.at[pl.ds(0, C)], sb0)
    pltpu.sync_copy(dst_hbm.at[pl.ds(0, C)], db0)
    start_chunk(1, sb1, db1, ss1, sd1)
    ptr, total = lax.fori_loop(0, NCHUNK, chunk_body,
                               (jnp.int32(0), jnp.int32(0)))

    for j in range(F // 16):
        pos = iota16 + j * 16
        vs = lsrc[pl.ds(j * 16, 16)]
        vd = ldst[pl.ds(j * 16, 16)]
        lsrc[pl.ds(j * 16, 16)] = jnp.where(pos >= ptr, 0, vs)
        ldst[pl.ds(j * 16, 16)] = jnp.where(pos >= ptr, lo + NPW, vd)
    foff = pl.multiple_of(w * LCAP + total, 8)
    pltpu.sync_copy(lsrc.at[pl.ds(0, F)], slist.at[pl.ds(foff, F)])
    pltpu.sync_copy(ldst.at[pl.ds(0, F)], dlist.at[pl.ds(foff, F)])

    cntv[...] = jnp.zeros((16,), jnp.int32) + (total // F + 1)
    pltpu.sync_copy(cntv, nblk.at[pl.ds(pl.multiple_of(w * 16, 8), 16)])


@functools.partial(
    pl.kernel,
    out_type=jax.ShapeDtypeStruct((N, D), jnp.float32),
    mesh=_mesh,
    compiler_params=_params,
    scratch_types=[
        pltpu.VMEM((F,), jnp.int32),
        pltpu.VMEM((F,), jnp.int32),
        pltpu.VMEM((F,), jnp.int32),
        pltpu.VMEM((F,), jnp.int32),
        pltpu.VMEM((F,), jnp.int32),
        pltpu.VMEM((F,), jnp.int32),
        pltpu.VMEM((F,), jnp.int32),
        pltpu.VMEM((F, D), jnp.float32),
        pltpu.VMEM((F, D), jnp.float32),
        pltpu.VMEM((F, D), jnp.float32),
        pltpu.VMEM((NPW + 1, D), jnp.float32),
        pltpu.VMEM((16,), jnp.int32),
        pltpu.SemaphoreType.DMA,
        pltpu.SemaphoreType.DMA,
        pltpu.SemaphoreType.DMA,
        pltpu.SemaphoreType.DMA,
        pltpu.SemaphoreType.DMA,
        pltpu.SemaphoreType.DMA,
        pltpu.SemaphoreType.DMA,
        pltpu.SemaphoreType.DMA,
        pltpu.SemaphoreType.DMA,
    ],
)
def _segmax(xin, slist, dlist, nblk, agg,
            sb0, sb1, sb2, db0, db1, db2, dcomp, rw0, rw1, rw2, acc, cntv,
            ss0, ss1, ss2, sd0, sd1, sd2, sg0, sg1, sg2):
    w = _wid()
    lo = w * NPW
    ninf = jnp.full((16,), -jnp.inf, jnp.float32)

    def init_body(r, _):
        for k in range(D // 16):
            acc[r, pl.ds(k * 16, 16)] = ninf
        return 0

    lax.fori_loop(0, NPW + 1, init_body, 0)

    pltpu.sync_copy(nblk.at[pl.ds(pl.multiple_of(w * 16, 8), 16)], cntv)
    nb = jnp.max(cntv[...])

    def start_idx(b, sb, db, ss, sd):
        off = pl.multiple_of(w * LCAP + b * F, 8)
        pltpu.async_copy(slist.at[pl.ds(off, F)], sb, ss)
        pltpu.async_copy(dlist.at[pl.ds(off, F)], db, sd)

    def wait_idx(sb, db, ss, sd):
        pltpu.make_async_copy(slist.at[pl.ds(0, F)], sb, ss).wait()
        pltpu.make_async_copy(dlist.at[pl.ds(0, F)], db, sd).wait()

    def stage(b, cur, nxt, nx2):
        sb, db, rw, ss, sd, sg = cur
        sb2_, db2_, rw2_, ss2_, sd2_, sg2_ = nx2

        @pl.when(b + 2 < nb)
        def _():
            wait_idx(sb2_, db2_, ss2_, sd2_)
            pltpu.async_copy(xin.at[sb2_], rw2_, sg2_)

        pltpu.make_async_copy(xin.at[sb], rw, sg).wait()
        for j in range(F // 16):
            dcomp[pl.ds(j * 16, 16)] = db[pl.ds(j * 16, 16)]

        @pl.when(b + 3 < nb)
        def _():
            start_idx(b + 3, sb, db, ss, sd)

        def group_body(g, _):
            dloc = dcomp[pl.ds(g * 16, 16)] - lo
            for l in range(16):
                e = g * 16 + l
                dl = dloc[l]
                avs = [acc[dl, pl.ds(k * 16, 16)] for k in range(D // 16)]
                rvs = [rw[e, pl.ds(k * 16, 16)] for k in range(D // 16)]
                for k in range(D // 16):
                    acc[dl, pl.ds(k * 16, 16)] = jnp.maximum(avs[k], rvs[k])
            return 0

        lax.fori_loop(0, F // 16, group_body, 0)

    bufs = [
        (sb0, db0, rw0, ss0, sd0, sg0),
        (sb1, db1, rw1, ss1, sd1, sg1),
        (sb2, db2, rw2, ss2, sd2, sg2),
    ]

    def block_body(b, _):
        for r in range(3):
            @pl.when(b % 3 == r)
            def _(r=r):
                stage(b, bufs[r], bufs[(r + 1) % 3], bufs[(r + 2) % 3])

        return 0

    off0 = pl.multiple_of(w * LCAP, 8)
    pltpu.sync_copy(slist.at[pl.ds(off0, F)], sb0)
    pltpu.sync_copy(dlist.at[pl.ds(off0, F)], db0)
    pltpu.async_copy(xin.at[sb0], rw0, sg0)

    @pl.when(nb > 1)
    def _():
        off1 = pl.multiple_of(w * LCAP + F, 8)
        pltpu.sync_copy(slist.at[pl.ds(off1, F)], sb1)
        pltpu.sync_copy(dlist.at[pl.ds(off1, F)], db1)
        pltpu.async_copy(xin.at[sb1], rw1, sg1)

    @pl.when(nb > 2)
    def _():
        start_idx(2, sb2, db2, ss2, sd2)

    lax.fori_loop(0, nb, block_body, 0)

    def out_body(r, _):
        vs = [acc[r, pl.ds(k * 16, 16)] for k in range(D // 16)]
        for k in range(D // 16):
            acc[r, pl.ds(k * 16, 16)] = jnp.where(vs[k] == -jnp.inf, 0.0,
                                                  vs[k])
        return 0

    lax.fori_loop(0, NPW, out_body, 0)

    @pl.when(w < NW - 1)
    def _():
        pltpu.sync_copy(acc.at[pl.ds(0, NPW), :], agg.at[pl.ds(lo, NPW), :])

    @pl.when(w == NW - 1)
    def _():
        pltpu.sync_copy(acc.at[pl.ds(0, N - (NW - 1) * NPW), :],
                        agg.at[pl.ds(lo, N - (NW - 1) * NPW), :])


_BR = 1000


def _mm(a, b):
    return lax.dot_general(a, b, (((1,), (0,)), ((), ())),
                           preferred_element_type=jnp.float32)


def _elu(z):
    return jnp.where(z > 0, z, jnp.exp(jnp.minimum(z, 0.0)) - 1.0)


def _dense_body(agg_ref, h_ref, wl_ref, wr_ref, b_ref, out_ref, *, res):
    z = _mm(agg_ref[...], wl_ref[...]) + _mm(h_ref[...], wr_ref[...]) + b_ref[...]
    h = _elu(z)
    if res:
        h = h + h_ref[...]
    out_ref[...] = h


def _final_body(agg_ref, h_ref, wl_ref, wr_ref, b_ref, wlin_ref, blin_ref,
                out_ref):
    z = _mm(agg_ref[...], wl_ref[...]) + _mm(h_ref[...], wr_ref[...]) + b_ref[...]
    h3 = _elu(z) + h_ref[...]
    t = _mm(h3, wlin_ref[...]) + blin_ref[...]
    out_ref[...] = 1.0 / (1.0 + jnp.exp(-t))


_row_spec = pl.BlockSpec((_BR, D), lambda i: (i, 0))
_w_spec = pl.BlockSpec((D, D), lambda i: (0, 0))
_b_spec = pl.BlockSpec((1, D), lambda i: (0, 0))


def _dense(agg, h, wl, wr, b, res):
    return pl.pallas_call(
        functools.partial(_dense_body, res=res),
        grid=(N // _BR,),
        in_specs=[_row_spec, _row_spec, _w_spec, _w_spec, _b_spec],
        out_specs=_row_spec,
        out_shape=jax.ShapeDtypeStruct((N, D), jnp.float32),
    )(agg, h, wl, wr, b.reshape(1, D))


def _final(agg, h, wl, wr, b, wlin_pad, blin):
    return pl.pallas_call(
        _final_body,
        grid=(N // _BR,),
        in_specs=[_row_spec, _row_spec, _w_spec, _w_spec, _b_spec, _w_spec,
                  _b_spec],
        out_specs=_row_spec,
        out_shape=jax.ShapeDtypeStruct((N, D), jnp.float32),
    )(agg, h, wl, wr, b.reshape(1, D), wlin_pad, blin)


def kernel(x, edge_index, W1l, W1r, b1, W2l, W2r, b2, W3l, W3r, b3, Wlin,
           blin):
    src = edge_index[0]
    dst = edge_index[1]
    wlin_pad = jnp.zeros((D, D), jnp.float32).at[:, :1].set(Wlin)
    blin_pad = jnp.zeros((1, D), jnp.float32) + blin

    slist, dlist, nblk = _phase0(src, dst)

    agg1 = _segmax(x, slist, dlist, nblk)
    h1 = _dense(agg1, x, W1l, W1r, b1, res=False)
    agg2 = _segmax(h1, slist, dlist, nblk)
    h2 = _dense(agg2, h1, W2l, W2r, b2, res=True)
    agg3 = _segmax(h2, slist, dlist, nblk)
    out = _final(agg3, h2, W3l, W3r, b3, wlin_pad, blin_pad)
    return out[:N, :1]
